# baseline (speedup 1.0000x reference)
import functools

import jax
import jax.numpy as jnp
import numpy as np
from jax import lax
from jax.experimental import pallas as pl
from jax.experimental.pallas import tpu as pltpu
from jax.experimental.pallas import tpu_sc as plsc

N = 10000
E = 160000
D = 256
H = 256
EPS = 1e-5

NC = 2
NS = 16
LANES = 16
HALF = H // 2

CHUNK = 40
EPT = 10240
EPAD = NS * EPT
TOTCH = EPT // CHUNK
SBC = 32
SB = TOTCH // SBC
NPAD = 10240
ROWS_PER_TILE = NPAD // NS

MM_BLK = 400


def _mm_split_body(x_ref, w_ref, o_ref):
    h = jnp.dot(x_ref[...], w_ref[...], preferred_element_type=jnp.float32)
    o_ref[0] = h[:, :HALF]
    o_ref[1] = h[:, HALF:]


def _mm_split(x, w):
    return pl.pallas_call(
        _mm_split_body,
        grid=(N // MM_BLK,),
        in_specs=[
            pl.BlockSpec((MM_BLK, D), lambda i: (i, 0)),
            pl.BlockSpec((D, H), lambda i: (0, 0)),
        ],
        out_specs=pl.BlockSpec((2, MM_BLK, HALF), lambda i: (0, i, 0)),
        out_shape=jax.ShapeDtypeStruct((2, N, HALF), jnp.float32),
    )(x, w)


def _mm_bn_body(a_ref, w_ref, s_ref, b_ref, o_ref):
    h0 = jnp.maximum(a_ref[0] * s_ref[0] + b_ref[0], 0.0)
    h1 = jnp.maximum(a_ref[1] * s_ref[1] + b_ref[1], 0.0)
    out = jnp.dot(h0, w_ref[:HALF, :], preferred_element_type=jnp.float32)
    out += jnp.dot(h1, w_ref[HALF:, :], preferred_element_type=jnp.float32)
    o_ref[0] = out[:, :HALF]
    o_ref[1] = out[:, HALF:]


def _mm_bn_split(a, w, scale, beta):
    return pl.pallas_call(
        _mm_bn_body,
        grid=(N // MM_BLK,),
        in_specs=[
            pl.BlockSpec((2, MM_BLK, HALF), lambda i: (0, i, 0)),
            pl.BlockSpec((H, H), lambda i: (0, 0)),
            pl.BlockSpec((2, 1, HALF), lambda i: (0, 0, 0)),
            pl.BlockSpec((2, 1, HALF), lambda i: (0, 0, 0)),
        ],
        out_specs=pl.BlockSpec((2, MM_BLK, HALF), lambda i: (0, i, 0)),
        out_shape=jax.ShapeDtypeStruct((2, N, HALF), jnp.float32),
    )(a, w, scale, beta)


def _sc_agg_body(hs_hbm, src_hbm, dst_hbm, w_hbm, zeros_hbm, out_hbm,
                 srcS, dst_sb, w_sb, r0_, r1_, r2_, r3_, acc_sh,
                 g0, g1, g2, g3, s0, s1, s2, s3):
    c = lax.axis_index("c")
    s = lax.axis_index("s")
    rows = (r0_, r1_, r2_, r3_)
    gsem = (g0, g1, g2, g3)
    ssem = (s0, s1, s2, s3)

    pltpu.sync_copy(zeros_hbm, acc_sh.at[pl.ds(s * ROWS_PER_TILE, ROWS_PER_TILE)])
    plsc.subcore_barrier()

    def mul_rows(p, x):
        r = rows[p]
        for g, base, lo in ((0, 0, 0), (1, 16, 0), (2, 24, 8)):
            wvec = w_sb[x, pl.ds(base, LANES)]
            for i16 in range(lo, LANES):
                wsplat = jnp.full((LANES,), wvec[i16], dtype=jnp.float32)
                i = base + i16
                for j in range(HALF // LANES):
                    sl = pl.ds(j * LANES, LANES)
                    r[i, sl] = r[i, sl] * wsplat

    def g_start(p, x):
        pltpu.async_copy(hs_hbm.at[srcS.at[x]], rows[p], gsem[p])

    def g_wait(p):
        pltpu.make_async_copy(hs_hbm.at[srcS.at[0]], rows[p], gsem[p]).wait()

    def s_start(p, x):
        pltpu.async_copy(rows[p], acc_sh.at[dst_sb.at[x]], ssem[p], add=True)

    def s_wait(p):
        pltpu.make_async_copy(rows[p], acc_sh.at[dst_sb.at[0]], ssem[p]).wait()

    pltpu.sync_copy(src_hbm.at[c, s, pl.ds(0, SBC + 8)], srcS)
    pltpu.sync_copy(dst_hbm.at[s, pl.ds(0, SBC)], dst_sb)
    pltpu.sync_copy(w_hbm.at[s, pl.ds(0, SBC)], w_sb)
    pltpu.sync_copy(zeros_hbm.at[pl.ds(0, CHUNK)], rows[2])
    pltpu.sync_copy(zeros_hbm.at[pl.ds(0, CHUNK)], rows[3])
    s_start(2, 0)
    s_start(3, 0)
    g_start(0, 0)
    g_start(1, 1)

    def steady(p, x):
        g_wait(p)
        mul_rows(p, x)
        s_start(p, x)
        pn = (p + 2) % 4
        s_wait(pn)
        g_start(pn, x + 2)

    def sb_body(sb, _):
        g_wait(0)
        g_wait(1)
        s_wait(2)
        s_wait(3)
        pltpu.sync_copy(src_hbm.at[c, s, pl.ds(sb * SBC, SBC + 8)], srcS)
        pltpu.sync_copy(dst_hbm.at[s, pl.ds(sb * SBC, SBC)], dst_sb)
        pltpu.sync_copy(w_hbm.at[s, pl.ds(sb * SBC, SBC)], w_sb)

        mul_rows(0, 0)
        s_start(0, 0)
        g_start(2, 2)
        mul_rows(1, 1)
        s_start(1, 1)
        g_start(3, 3)

        def quad(q, _):
            x = 2 + 4 * q
            steady(2, x)
            steady(3, x + 1)
            steady(0, x + 2)
            steady(1, x + 3)
            return ()

        lax.fori_loop(0, (SBC - 4) // 4, quad, ())

        steady(2, SBC - 2)
        steady(3, SBC - 1)
        return ()

    lax.fori_loop(0, SB, sb_body, ())

    g_wait(0)
    g_wait(1)
    s_wait(2)
    s_wait(3)

    plsc.subcore_barrier()

    r0 = s * ROWS_PER_TILE
    pltpu.sync_copy(acc_sh.at[pl.ds(r0, ROWS_PER_TILE)],
                    out_hbm.at[pl.ds(c * NPAD + r0, ROWS_PER_TILE)])


@functools.partial(
    pl.kernel,
    out_type=jax.ShapeDtypeStruct((2 * NPAD, HALF), jnp.float32),
    mesh=plsc.VectorSubcoreMesh(core_axis_name="c", subcore_axis_name="s",
                                num_cores=NC, num_subcores=NS),
    scratch_types=[
        pltpu.VMEM((SBC + 8, CHUNK), jnp.int32),
        pltpu.VMEM((SBC, CHUNK), jnp.int32),
        pltpu.VMEM((SBC, CHUNK), jnp.float32),
        pltpu.VMEM((CHUNK, HALF), jnp.float32),
        pltpu.VMEM((CHUNK, HALF), jnp.float32),
        pltpu.VMEM((CHUNK, HALF), jnp.float32),
        pltpu.VMEM((CHUNK, HALF), jnp.float32),
        pltpu.VMEM_SHARED((NPAD, HALF), jnp.float32),
        pltpu.SemaphoreType.DMA,
        pltpu.SemaphoreType.DMA,
        pltpu.SemaphoreType.DMA,
        pltpu.SemaphoreType.DMA,
        pltpu.SemaphoreType.DMA,
        pltpu.SemaphoreType.DMA,
        pltpu.SemaphoreType.DMA,
        pltpu.SemaphoreType.DMA,
    ],
)
def _sc_agg(hs_hbm, src_hbm, dst_hbm, w_hbm, zeros_hbm, out_hbm,
            srcS, dst_sb, w_sb, r0_, r1_, r2_, r3_, acc_sh,
            g0, g1, g2, g3, s0, s1, s2, s3):
    _sc_agg_body(hs_hbm, src_hbm, dst_hbm, w_hbm, zeros_hbm, out_hbm,
                 srcS, dst_sb, w_sb, r0_, r1_, r2_, r3_, acc_sh,
                 g0, g1, g2, g3, s0, s1, s2, s3)


def kernel(x, edge_index, edge_attr, batch, W1, W2, gamma1, beta1):
    src = edge_index[0]
    dst = edge_index[1]
    pad = EPAD - E
    srcp = jnp.concatenate([src, jnp.zeros((pad,), jnp.int32)])
    dstp = jnp.concatenate([dst, jnp.zeros((pad,), jnp.int32)])
    wp = jnp.concatenate([edge_attr, jnp.zeros((pad,), jnp.float32)])
    srh = srcp.reshape(NS, TOTCH, CHUNK)
    srh = jnp.concatenate([srh, jnp.zeros((NS, 8, CHUNK), jnp.int32)], axis=1)
    src4 = jnp.stack([srh, srh + N])
    dst3 = dstp.reshape(NS, TOTCH, CHUNK)
    w3 = wp.reshape(NS, TOTCH, CHUNK)
    zeros = jnp.zeros((ROWS_PER_TILE, HALF), jnp.float32)

    scale = (gamma1 * np.float32(1.0 / np.sqrt(1.0 + EPS))).reshape(2, 1, HALF)
    beta = beta1.reshape(2, 1, HALF)

    h1 = _mm_split(x, W1)
    a1 = _sc_agg(h1.reshape(2 * N, HALF), src4, dst3, w3, zeros)
    h2 = _mm_bn_split(a1.reshape(2, NPAD, HALF), W2, scale, beta)
    a2 = _sc_agg(h2.reshape(2 * N, HALF), src4, dst3, w3, zeros)
    r = a2.reshape(2, NPAD, HALF)
    return jnp.concatenate([r[0, :N], r[1, :N]], axis=1)
